# Initial kernel scaffold; baseline (speedup 1.0000x reference)
import functools

import jax
import jax.numpy as jnp
from jax import lax
from jax.experimental import pallas as pl
from jax.experimental.pallas import tpu as pltpu
from jax.experimental.pallas import tpu_sc as plsc

N_NODES = 10000
N_EDGES = 320000
D = 128
H = 256

NC = 2
NS = 16
NW = NC * NS
EPW = N_EDGES // NW
C = 80
CHUNKS = EPW // C
RPT = N_NODES // NS


def _sc_gather(x, src):
    mesh = plsc.VectorSubcoreMesh(core_axis_name="c", subcore_axis_name="s")

    @functools.partial(
        pl.kernel,
        out_type=jax.ShapeDtypeStruct((N_EDGES, D), x.dtype),
        mesh=mesh,
        scratch_types=[
            pltpu.VMEM((C,), jnp.int32),
            pltpu.VMEM((C, D), x.dtype),
            pltpu.SemaphoreType.DMA,
        ],
    )
    def k(x_hbm, src_hbm, out_hbm, idx_v, rows_v, sem):
        wid = lax.axis_index("s") * NC + lax.axis_index("c")
        base = wid * EPW

        def chunk(c, carry):
            off = base + c * C
            pltpu.sync_copy(src_hbm.at[pl.ds(off, C)], idx_v)
            pltpu.async_copy(x_hbm.at[idx_v], rows_v, sem).wait()
            pltpu.sync_copy(rows_v, out_hbm.at[pl.ds(off, C)])
            return carry

        lax.fori_loop(0, CHUNKS, chunk, 0)

    return k(x, src)


def _sc_scatter(msgs, dst, zeros):
    mesh = plsc.VectorSubcoreMesh(core_axis_name="c", subcore_axis_name="s")

    @functools.partial(
        pl.kernel,
        out_type=jax.ShapeDtypeStruct((NC, N_NODES, D), jnp.float32),
        mesh=mesh,
        scratch_types=[
            pltpu.VMEM((C,), jnp.int32),
            pltpu.VMEM((C, D), jnp.float32),
            pltpu.VMEM_SHARED((N_NODES, D), jnp.float32),
        ],
    )
    def k(msgs_hbm, dst_hbm, z_hbm, out_hbm, idx_v, m_v, fsh):
        cid = lax.axis_index("c")
        sid = lax.axis_index("s")
        wid = sid * NC + cid
        r0 = sid * RPT
        pltpu.sync_copy(z_hbm.at[pl.ds(r0, RPT)], fsh.at[pl.ds(r0, RPT)])
        plsc.subcore_barrier()

        base = wid * EPW

        def chunk(c, carry):
            off = base + c * C
            pltpu.sync_copy(dst_hbm.at[pl.ds(off, C)], idx_v)
            pltpu.sync_copy(msgs_hbm.at[pl.ds(off, C)], m_v)
            pltpu.sync_copy(m_v, fsh.at[idx_v], add=True)
            return carry

        lax.fori_loop(0, CHUNKS, chunk, 0)
        plsc.subcore_barrier()
        pltpu.sync_copy(fsh.at[pl.ds(r0, RPT)], out_hbm.at[cid, pl.ds(r0, RPT)])

    return k(msgs, dst, zeros)


ER = 1280


def _tc_edge(ea, xg, Wu1, bu1, Wu2, bu2):
    def body(ea_ref, xg_ref, w1_ref, b1_ref, w2_ref, b2_ref, edges_ref, msgs_ref):
        a = ea_ref[...]
        h = jnp.tanh(jnp.dot(a, w1_ref[...], preferred_element_type=jnp.float32)
                     + b1_ref[...])
        e = a + jnp.dot(h, w2_ref[...], preferred_element_type=jnp.float32) + b2_ref[...]
        edges_ref[...] = e
        msgs_ref[...] = e * xg_ref[...].astype(jnp.float32)

    grid = (N_EDGES // ER,)
    return pl.pallas_call(
        body,
        grid=grid,
        in_specs=[
            pl.BlockSpec((ER, D), lambda i: (i, 0)),
            pl.BlockSpec((ER, D), lambda i: (i, 0)),
            pl.BlockSpec((D, H), lambda i: (0, 0)),
            pl.BlockSpec((1, H), lambda i: (0, 0)),
            pl.BlockSpec((H, D), lambda i: (0, 0)),
            pl.BlockSpec((1, D), lambda i: (0, 0)),
        ],
        out_specs=[
            pl.BlockSpec((ER, D), lambda i: (i, 0)),
            pl.BlockSpec((ER, D), lambda i: (i, 0)),
        ],
        out_shape=[
            jax.ShapeDtypeStruct((N_EDGES, D), jnp.float32),
            jax.ShapeDtypeStruct((N_EDGES, D), jnp.float32),
        ],
    )(ea, xg, Wu1, bu1, Wu2, bu2)


NR = 2000


def _tc_node(x, fp, Wg1, bg1, Wg2, bg2):
    def body(x_ref, fp_ref, w1_ref, b1_ref, w2_ref, b2_ref, out_ref):
        f = fp_ref[0] + fp_ref[1]
        h = jnp.tanh(jnp.dot(f, w1_ref[...], preferred_element_type=jnp.float32)
                     + b1_ref[...])
        out_ref[...] = x_ref[...] + jnp.dot(
            h, w2_ref[...], preferred_element_type=jnp.float32) + b2_ref[...]

    grid = (N_NODES // NR,)
    return pl.pallas_call(
        body,
        grid=grid,
        in_specs=[
            pl.BlockSpec((NR, D), lambda i: (i, 0)),
            pl.BlockSpec((NC, NR, D), lambda i: (0, i, 0)),
            pl.BlockSpec((D, H), lambda i: (0, 0)),
            pl.BlockSpec((1, H), lambda i: (0, 0)),
            pl.BlockSpec((H, D), lambda i: (0, 0)),
            pl.BlockSpec((1, D), lambda i: (0, 0)),
        ],
        out_specs=pl.BlockSpec((NR, D), lambda i: (i, 0)),
        out_shape=jax.ShapeDtypeStruct((N_NODES, D), jnp.float32),
    )(x, fp, Wg1, bg1, Wg2, bg2)


def kernel(x, edge_attr, edge_index, Wu1, bu1, Wu2, bu2, Wg1, bg1, Wg2, bg2):
    src = edge_index[0].astype(jnp.int32)
    dst = edge_index[1].astype(jnp.int32)

    xg = _sc_gather(x, src)
    edges, msgs = _tc_edge(edge_attr, xg, Wu1, bu1.reshape(1, H), Wu2,
                           bu2.reshape(1, D))
    zeros = jnp.zeros((N_NODES, D), jnp.float32)
    fp = _sc_scatter(msgs, dst, zeros)
    nodes = _tc_node(x, fp, Wg1, bg1.reshape(1, H), Wg2, bg2.reshape(1, D))
    return nodes, edges


# speedup vs baseline: 2.2589x; 2.2589x over previous
import functools

import jax
import jax.numpy as jnp
from jax import lax
from jax.experimental import pallas as pl
from jax.experimental.pallas import tpu as pltpu
from jax.experimental.pallas import tpu_sc as plsc

N_NODES = 10000
N_EDGES = 320000
D = 128
H = 256

NC = 2
NS = 16
NW = NC * NS
EPW = N_EDGES // NW
C = 80
CHUNKS = EPW // C
RPT = 624
TAIL0 = RPT * NS
TAIL = N_NODES - TAIL0


def _sc_gather(x, src):
    mesh = plsc.VectorSubcoreMesh(core_axis_name="c", subcore_axis_name="s")

    @functools.partial(
        pl.kernel,
        out_type=jax.ShapeDtypeStruct((N_EDGES, D), x.dtype),
        mesh=mesh,
        scratch_types=[
            pltpu.VMEM((C,), jnp.int32),
            pltpu.VMEM((C, D), x.dtype),
            pltpu.SemaphoreType.DMA,
        ],
    )
    def k(x_hbm, src_hbm, out_hbm, idx_v, rows_v, sem):
        wid = lax.axis_index("s") * NC + lax.axis_index("c")
        base = wid * EPW

        def chunk(c, carry):
            off = base + c * C
            pltpu.sync_copy(src_hbm.at[pl.ds(off, C)], idx_v)
            pltpu.async_copy(x_hbm.at[idx_v], rows_v, sem).wait()
            pltpu.sync_copy(rows_v, out_hbm.at[pl.ds(off, C)])
            return carry

        lax.fori_loop(0, CHUNKS, chunk, 0)

    return k(x, src)


def _sc_scatter(msgs, dst, zeros):
    mesh = plsc.VectorSubcoreMesh(core_axis_name="c", subcore_axis_name="s")

    @functools.partial(
        pl.kernel,
        out_type=jax.ShapeDtypeStruct((NC, N_NODES, D), jnp.float32),
        mesh=mesh,
        scratch_types=[
            pltpu.VMEM((C,), jnp.int32),
            pltpu.VMEM((C, D), jnp.float32),
            pltpu.VMEM_SHARED((N_NODES, D), jnp.float32),
        ],
    )
    def k(msgs_hbm, dst_hbm, z_hbm, out_hbm, idx_v, m_v, fsh):
        cid = lax.axis_index("c")
        sid = lax.axis_index("s")
        wid = sid * NC + cid
        r0 = sid * RPT
        pltpu.sync_copy(z_hbm.at[pl.ds(r0, RPT)], fsh.at[pl.ds(r0, RPT)])

        @pl.when(sid == NS - 1)
        def _():
            pltpu.sync_copy(z_hbm.at[pl.ds(TAIL0, TAIL)], fsh.at[pl.ds(TAIL0, TAIL)])

        plsc.subcore_barrier()

        base = wid * EPW

        def chunk(c, carry):
            off = base + c * C
            pltpu.sync_copy(dst_hbm.at[pl.ds(off, C)], idx_v)
            pltpu.sync_copy(msgs_hbm.at[pl.ds(off, C)], m_v)
            pltpu.sync_copy(m_v, fsh.at[idx_v], add=True)
            return carry

        lax.fori_loop(0, CHUNKS, chunk, 0)
        plsc.subcore_barrier()
        pltpu.sync_copy(fsh.at[pl.ds(r0, RPT)], out_hbm.at[cid, pl.ds(r0, RPT)])

        @pl.when(sid == NS - 1)
        def _():
            pltpu.sync_copy(fsh.at[pl.ds(TAIL0, TAIL)],
                            out_hbm.at[cid, pl.ds(TAIL0, TAIL)])

    return k(msgs, dst, zeros)


ER = 1280


def _tc_edge(ea, xg, Wu1, bu1, Wu2, bu2):
    def body(ea_ref, xg_ref, w1_ref, b1_ref, w2_ref, b2_ref, edges_ref, msgs_ref):
        a = ea_ref[...]
        h = jnp.tanh(jnp.dot(a, w1_ref[...], preferred_element_type=jnp.float32)
                     + b1_ref[...])
        e = a + jnp.dot(h, w2_ref[...], preferred_element_type=jnp.float32) + b2_ref[...]
        edges_ref[...] = e
        msgs_ref[...] = e * xg_ref[...].astype(jnp.float32)

    grid = (N_EDGES // ER,)
    return pl.pallas_call(
        body,
        grid=grid,
        in_specs=[
            pl.BlockSpec((ER, D), lambda i: (i, 0)),
            pl.BlockSpec((ER, D), lambda i: (i, 0)),
            pl.BlockSpec((D, H), lambda i: (0, 0)),
            pl.BlockSpec((1, H), lambda i: (0, 0)),
            pl.BlockSpec((H, D), lambda i: (0, 0)),
            pl.BlockSpec((1, D), lambda i: (0, 0)),
        ],
        out_specs=[
            pl.BlockSpec((ER, D), lambda i: (i, 0)),
            pl.BlockSpec((ER, D), lambda i: (i, 0)),
        ],
        out_shape=[
            jax.ShapeDtypeStruct((N_EDGES, D), jnp.float32),
            jax.ShapeDtypeStruct((N_EDGES, D), jnp.float32),
        ],
    )(ea, xg, Wu1, bu1, Wu2, bu2)


NR = 2000


def _tc_node(x, fp, Wg1, bg1, Wg2, bg2):
    def body(x_ref, fp_ref, w1_ref, b1_ref, w2_ref, b2_ref, out_ref):
        f = fp_ref[0] + fp_ref[1]
        h = jnp.tanh(jnp.dot(f, w1_ref[...], preferred_element_type=jnp.float32)
                     + b1_ref[...])
        out_ref[...] = x_ref[...] + jnp.dot(
            h, w2_ref[...], preferred_element_type=jnp.float32) + b2_ref[...]

    grid = (N_NODES // NR,)
    return pl.pallas_call(
        body,
        grid=grid,
        in_specs=[
            pl.BlockSpec((NR, D), lambda i: (i, 0)),
            pl.BlockSpec((NC, NR, D), lambda i: (0, i, 0)),
            pl.BlockSpec((D, H), lambda i: (0, 0)),
            pl.BlockSpec((1, H), lambda i: (0, 0)),
            pl.BlockSpec((H, D), lambda i: (0, 0)),
            pl.BlockSpec((1, D), lambda i: (0, 0)),
        ],
        out_specs=pl.BlockSpec((NR, D), lambda i: (i, 0)),
        out_shape=jax.ShapeDtypeStruct((N_NODES, D), jnp.float32),
    )(x, fp, Wg1, bg1, Wg2, bg2)


def kernel(x, edge_attr, edge_index, Wu1, bu1, Wu2, bu2, Wg1, bg1, Wg2, bg2):
    src = edge_index[0].astype(jnp.int32)
    dst = edge_index[1].astype(jnp.int32)

    xg = _sc_gather(x, src)
    edges, msgs = _tc_edge(edge_attr, xg, Wu1, bu1.reshape(1, H), Wu2,
                           bu2.reshape(1, D))
    zeros = jnp.zeros((N_NODES, D), jnp.float32)
    fp = _sc_scatter(msgs, dst, zeros)
    nodes = _tc_node(x, fp, Wg1, bg1.reshape(1, H), Wg2, bg2.reshape(1, D))
    return nodes, edges


# speedup vs baseline: 2.2639x; 1.0022x over previous
import functools

import jax
import jax.numpy as jnp
from jax import lax
from jax.experimental import pallas as pl
from jax.experimental.pallas import tpu as pltpu
from jax.experimental.pallas import tpu_sc as plsc

N_NODES = 10000
N_EDGES = 320000
D = 128
H = 256

NC = 2
NS = 16
NW = NC * NS
EPW = N_EDGES // NW
C = 80
CHUNKS = EPW // C
RPT = 624
TAIL0 = RPT * NS
TAIL = N_NODES - TAIL0


def _sc_gather(x, src):
    mesh = plsc.VectorSubcoreMesh(core_axis_name="c", subcore_axis_name="s")

    @functools.partial(
        pl.kernel,
        out_type=jax.ShapeDtypeStruct((N_EDGES, D), x.dtype),
        mesh=mesh,
        scratch_types=[
            pltpu.VMEM((C,), jnp.int32),
            pltpu.VMEM((C, D), x.dtype),
            pltpu.SemaphoreType.DMA,
        ],
    )
    def k(x_hbm, src_hbm, out_hbm, idx_v, rows_v, sem):
        wid = lax.axis_index("s") * NC + lax.axis_index("c")
        base = wid * EPW

        def chunk(c, carry):
            off = base + c * C
            pltpu.sync_copy(src_hbm.at[pl.ds(off, C)], idx_v)
            pltpu.async_copy(x_hbm.at[idx_v], rows_v, sem).wait()
            pltpu.sync_copy(rows_v, out_hbm.at[pl.ds(off, C)])
            return carry

        lax.fori_loop(0, CHUNKS, chunk, 0)

    return k(x, src)


def _sc_scatter(msgs, dst, zeros):
    mesh = plsc.VectorSubcoreMesh(core_axis_name="c", subcore_axis_name="s")

    @functools.partial(
        pl.kernel,
        out_type=jax.ShapeDtypeStruct((NC, N_NODES, D), jnp.float32),
        mesh=mesh,
        scratch_types=[
            pltpu.VMEM((C,), jnp.int32),
            pltpu.VMEM((C, D), jnp.float32),
            pltpu.VMEM_SHARED((N_NODES, D), jnp.float32),
        ],
    )
    def k(msgs_hbm, dst_hbm, z_hbm, out_hbm, idx_v, m_v, fsh):
        cid = lax.axis_index("c")
        sid = lax.axis_index("s")
        wid = sid * NC + cid
        r0 = sid * RPT
        pltpu.sync_copy(z_hbm.at[pl.ds(r0, RPT)], fsh.at[pl.ds(r0, RPT)])

        @pl.when(sid == NS - 1)
        def _():
            pltpu.sync_copy(z_hbm.at[pl.ds(TAIL0, TAIL)], fsh.at[pl.ds(TAIL0, TAIL)])

        plsc.subcore_barrier()

        base = wid * EPW

        def chunk(c, carry):
            off = base + c * C
            pltpu.sync_copy(dst_hbm.at[pl.ds(off, C)], idx_v)
            pltpu.sync_copy(msgs_hbm.at[pl.ds(off, C)], m_v)
            pltpu.sync_copy(m_v, fsh.at[idx_v], add=True)
            return carry

        lax.fori_loop(0, CHUNKS, chunk, 0)
        plsc.subcore_barrier()
        pltpu.sync_copy(fsh.at[pl.ds(r0, RPT)], out_hbm.at[cid, pl.ds(r0, RPT)])

        @pl.when(sid == NS - 1)
        def _():
            pltpu.sync_copy(fsh.at[pl.ds(TAIL0, TAIL)],
                            out_hbm.at[cid, pl.ds(TAIL0, TAIL)])

    return k(msgs, dst, zeros)


ER = 1280


def _tc_edge(ea, xg, Wu1, bu1, Wu2, bu2):
    def body(ea_ref, xg_ref, w1_ref, b1_ref, w2_ref, b2_ref, edges_ref, msgs_ref):
        a = ea_ref[...]
        h = jnp.tanh(jnp.dot(a.astype(jnp.bfloat16), w1_ref[...].astype(jnp.bfloat16),
                             preferred_element_type=jnp.float32) + b1_ref[...])
        e = a + jnp.dot(h.astype(jnp.bfloat16), w2_ref[...].astype(jnp.bfloat16),
                        preferred_element_type=jnp.float32) + b2_ref[...]
        edges_ref[...] = e
        msgs_ref[...] = e * xg_ref[...].astype(jnp.float32)

    grid = (N_EDGES // ER,)
    return pl.pallas_call(
        body,
        grid=grid,
        in_specs=[
            pl.BlockSpec((ER, D), lambda i: (i, 0)),
            pl.BlockSpec((ER, D), lambda i: (i, 0)),
            pl.BlockSpec((D, H), lambda i: (0, 0)),
            pl.BlockSpec((1, H), lambda i: (0, 0)),
            pl.BlockSpec((H, D), lambda i: (0, 0)),
            pl.BlockSpec((1, D), lambda i: (0, 0)),
        ],
        out_specs=[
            pl.BlockSpec((ER, D), lambda i: (i, 0)),
            pl.BlockSpec((ER, D), lambda i: (i, 0)),
        ],
        out_shape=[
            jax.ShapeDtypeStruct((N_EDGES, D), jnp.float32),
            jax.ShapeDtypeStruct((N_EDGES, D), jnp.float32),
        ],
    )(ea, xg, Wu1, bu1, Wu2, bu2)


NR = 2000


def _tc_node(x, fp, Wg1, bg1, Wg2, bg2):
    def body(x_ref, fp_ref, w1_ref, b1_ref, w2_ref, b2_ref, out_ref):
        f = fp_ref[0] + fp_ref[1]
        h = jnp.tanh(jnp.dot(f.astype(jnp.bfloat16), w1_ref[...].astype(jnp.bfloat16),
                             preferred_element_type=jnp.float32) + b1_ref[...])
        out_ref[...] = x_ref[...] + jnp.dot(
            h.astype(jnp.bfloat16), w2_ref[...].astype(jnp.bfloat16),
            preferred_element_type=jnp.float32) + b2_ref[...]

    grid = (N_NODES // NR,)
    return pl.pallas_call(
        body,
        grid=grid,
        in_specs=[
            pl.BlockSpec((NR, D), lambda i: (i, 0)),
            pl.BlockSpec((NC, NR, D), lambda i: (0, i, 0)),
            pl.BlockSpec((D, H), lambda i: (0, 0)),
            pl.BlockSpec((1, H), lambda i: (0, 0)),
            pl.BlockSpec((H, D), lambda i: (0, 0)),
            pl.BlockSpec((1, D), lambda i: (0, 0)),
        ],
        out_specs=pl.BlockSpec((NR, D), lambda i: (i, 0)),
        out_shape=jax.ShapeDtypeStruct((N_NODES, D), jnp.float32),
    )(x, fp, Wg1, bg1, Wg2, bg2)


def kernel(x, edge_attr, edge_index, Wu1, bu1, Wu2, bu2, Wg1, bg1, Wg2, bg2):
    src = edge_index[0].astype(jnp.int32)
    dst = edge_index[1].astype(jnp.int32)

    xg = _sc_gather(x, src)
    edges, msgs = _tc_edge(edge_attr, xg, Wu1, bu1.reshape(1, H), Wu2,
                           bu2.reshape(1, D))
    zeros = jnp.zeros((N_NODES, D), jnp.float32)
    fp = _sc_scatter(msgs, dst, zeros)
    nodes = _tc_node(x, fp, Wg1, bg1.reshape(1, H), Wg2, bg2.reshape(1, D))
    return nodes, edges


# speedup vs baseline: 4.0325x; 1.7812x over previous
import functools

import jax
import jax.numpy as jnp
from jax import lax
from jax.experimental import pallas as pl
from jax.experimental.pallas import tpu as pltpu
from jax.experimental.pallas import tpu_sc as plsc

N_NODES = 10000
N_EDGES = 320000
D = 128
H = 256

NC = 2
NS = 16
NW = NC * NS
EPW = N_EDGES // NW
C = 40
CHUNKS = EPW // C
RPT = 624
TAIL0 = RPT * NS
TAIL = N_NODES - TAIL0

NB = 4
LA = 2
BODY = (CHUNKS // NB) * NB


def _sc_fused(edges, x, src, dst, zeros):
    mesh = plsc.VectorSubcoreMesh(core_axis_name="c", subcore_axis_name="s")

    scratch = (
        [pltpu.VMEM((C,), jnp.int32) for _ in range(2 * NB)]
        + [pltpu.VMEM((C, D), jnp.float32) for _ in range(2 * NB)]
        + [pltpu.VMEM_SHARED((N_NODES, D), jnp.float32)]
        + [pltpu.SemaphoreType.DMA for _ in range(3 * NB)]
    )

    @functools.partial(
        pl.kernel,
        out_type=jax.ShapeDtypeStruct((NC, N_NODES, D), jnp.float32),
        mesh=mesh,
        scratch_types=scratch,
    )
    def k(edges_hbm, x_hbm, src_hbm, dst_hbm, z_hbm, out_hbm, *scr):
        srcv = scr[0:NB]
        dstv = scr[NB:2 * NB]
        ev = scr[2 * NB:3 * NB]
        xv = scr[3 * NB:4 * NB]
        fsh = scr[4 * NB]
        sem_i = scr[4 * NB + 1: 4 * NB + 1 + NB]
        sem_g = scr[4 * NB + 1 + NB: 4 * NB + 1 + 2 * NB]
        sem_d = scr[4 * NB + 1 + 2 * NB: 4 * NB + 1 + 3 * NB]

        cid = lax.axis_index("c")
        sid = lax.axis_index("s")
        wid = sid * NC + cid
        base = wid * EPW

        r0 = sid * RPT
        pltpu.sync_copy(z_hbm.at[pl.ds(r0, RPT)], fsh.at[pl.ds(r0, RPT)])

        @pl.when(sid == NS - 1)
        def _():
            pltpu.sync_copy(z_hbm.at[pl.ds(TAIL0, TAIL)], fsh.at[pl.ds(TAIL0, TAIL)])

        plsc.subcore_barrier()

        def phase1(kk, s):
            off = base + kk * C
            pltpu.async_copy(src_hbm.at[pl.ds(off, C)], srcv[s], sem_i[s])
            pltpu.async_copy(dst_hbm.at[pl.ds(off, C)], dstv[s], sem_i[s])
            pltpu.async_copy(edges_hbm.at[pl.ds(off, C)], ev[s], sem_g[s])

        def drain_i(s):
            pltpu.make_async_copy(src_hbm.at[pl.ds(0, C)], srcv[s], sem_i[s]).wait()
            pltpu.make_async_copy(src_hbm.at[pl.ds(0, C)], dstv[s], sem_i[s]).wait()

        def drain_g(s):
            pltpu.make_async_copy(edges_hbm.at[pl.ds(0, C)], ev[s], sem_g[s]).wait()
            pltpu.make_async_copy(edges_hbm.at[pl.ds(0, C)], xv[s], sem_g[s]).wait()

        def drain_d(s):
            pltpu.make_async_copy(edges_hbm.at[pl.ds(0, C)], ev[s], sem_d[s]).wait()

        def gather(s):
            pltpu.async_copy(x_hbm.at[srcv[s]], xv[s], sem_g[s])

        def position(kk, b, static):
            s_pre = (b + LA) % NB
            s_nxt = (b + 1) % NB
            when = (lambda c: pl.when(bool(c))) if static else pl.when

            @when(kk >= NB - LA)
            def _():
                drain_d(s_pre)

            @when(kk + LA < CHUNKS)
            def _():
                phase1(kk + LA, s_pre)

            @when(kk + 1 < CHUNKS)
            def _():
                drain_i(s_nxt)
                gather(s_nxt)

            drain_g(b)

            @plsc.parallel_loop(0, C, 1, unroll=2)
            def _(i):
                for j in range(D // 16):
                    sl = pl.ds(j * 16, 16)
                    ev[b][i, sl] = ev[b][i, sl] * xv[b][i, sl]

            pltpu.async_copy(ev[b], fsh.at[dstv[b]], sem_d[b], add=True)

        phase1(0, 0)
        phase1(1, 1)
        drain_i(0)
        gather(0)

        def outer(go, carry):
            for b in range(NB):
                position(go + b, b, static=False)
            return carry

        lax.fori_loop(0, BODY // NB, lambda g, c: outer(g * NB, c), 0)

        for kk in range(BODY, CHUNKS):
            position(kk, kk % NB, static=True)
        for kk in range(CHUNKS - LA, CHUNKS):
            drain_d(kk % NB)

        plsc.subcore_barrier()
        pltpu.sync_copy(fsh.at[pl.ds(r0, RPT)], out_hbm.at[cid, pl.ds(r0, RPT)])

        @pl.when(sid == NS - 1)
        def _():
            pltpu.sync_copy(fsh.at[pl.ds(TAIL0, TAIL)],
                            out_hbm.at[cid, pl.ds(TAIL0, TAIL)])

    return k(edges, x, src, dst, zeros)


ER = 1280


def _tc_edge(ea, Wu1, bu1, Wu2, bu2):
    def body(ea_ref, w1_ref, b1_ref, w2_ref, b2_ref, edges_ref):
        a = ea_ref[...]
        h = jnp.tanh(jnp.dot(a.astype(jnp.bfloat16), w1_ref[...].astype(jnp.bfloat16),
                             preferred_element_type=jnp.float32) + b1_ref[...])
        edges_ref[...] = a + jnp.dot(
            h.astype(jnp.bfloat16), w2_ref[...].astype(jnp.bfloat16),
            preferred_element_type=jnp.float32) + b2_ref[...]

    grid = (N_EDGES // ER,)
    return pl.pallas_call(
        body,
        grid=grid,
        in_specs=[
            pl.BlockSpec((ER, D), lambda i: (i, 0)),
            pl.BlockSpec((D, H), lambda i: (0, 0)),
            pl.BlockSpec((1, H), lambda i: (0, 0)),
            pl.BlockSpec((H, D), lambda i: (0, 0)),
            pl.BlockSpec((1, D), lambda i: (0, 0)),
        ],
        out_specs=pl.BlockSpec((ER, D), lambda i: (i, 0)),
        out_shape=jax.ShapeDtypeStruct((N_EDGES, D), jnp.float32),
    )(ea, Wu1, bu1, Wu2, bu2)


NR = 2000


def _tc_node(x, fp, Wg1, bg1, Wg2, bg2):
    def body(x_ref, fp_ref, w1_ref, b1_ref, w2_ref, b2_ref, out_ref):
        f = fp_ref[0] + fp_ref[1]
        h = jnp.tanh(jnp.dot(f.astype(jnp.bfloat16), w1_ref[...].astype(jnp.bfloat16),
                             preferred_element_type=jnp.float32) + b1_ref[...])
        out_ref[...] = x_ref[...] + jnp.dot(
            h.astype(jnp.bfloat16), w2_ref[...].astype(jnp.bfloat16),
            preferred_element_type=jnp.float32) + b2_ref[...]

    grid = (N_NODES // NR,)
    return pl.pallas_call(
        body,
        grid=grid,
        in_specs=[
            pl.BlockSpec((NR, D), lambda i: (i, 0)),
            pl.BlockSpec((NC, NR, D), lambda i: (0, i, 0)),
            pl.BlockSpec((D, H), lambda i: (0, 0)),
            pl.BlockSpec((1, H), lambda i: (0, 0)),
            pl.BlockSpec((H, D), lambda i: (0, 0)),
            pl.BlockSpec((1, D), lambda i: (0, 0)),
        ],
        out_specs=pl.BlockSpec((NR, D), lambda i: (i, 0)),
        out_shape=jax.ShapeDtypeStruct((N_NODES, D), jnp.float32),
    )(x, fp, Wg1, bg1, Wg2, bg2)


def kernel(x, edge_attr, edge_index, Wu1, bu1, Wu2, bu2, Wg1, bg1, Wg2, bg2):
    src = edge_index[0].astype(jnp.int32)
    dst = edge_index[1].astype(jnp.int32)

    edges = _tc_edge(edge_attr, Wu1, bu1.reshape(1, H), Wu2, bu2.reshape(1, D))
    zeros = jnp.zeros((N_NODES, D), jnp.float32)
    fp = _sc_fused(edges, x, src, dst, zeros)
    nodes = _tc_node(x, fp, Wg1, bg1.reshape(1, H), Wg2, bg2.reshape(1, D))
    return nodes, edges


# speedup vs baseline: 4.7741x; 1.1839x over previous
import functools

import jax
import jax.numpy as jnp
from jax import lax
from jax.experimental import pallas as pl
from jax.experimental.pallas import tpu as pltpu
from jax.experimental.pallas import tpu_sc as plsc

N_NODES = 10000
N_EDGES = 320000
D = 128
H = 256

NC = 2
NS = 16
NW = NC * NS
EPW = N_EDGES // NW
C = 40
CHUNKS = EPW // C
RPT = 624
TAIL0 = RPT * NS
TAIL = N_NODES - TAIL0

NB = 4
LA = 2
BODY = (CHUNKS // NB) * NB


def _sc_fused(edges, x, src, dst, zeros):
    mesh = plsc.VectorSubcoreMesh(core_axis_name="c", subcore_axis_name="s")

    scratch = (
        [pltpu.VMEM((C,), jnp.int32) for _ in range(2 * NB)]
        + [pltpu.VMEM((C, D), jnp.float32) for _ in range(2 * NB)]
        + [pltpu.VMEM_SHARED((N_NODES, D), jnp.float32)]
        + [pltpu.SemaphoreType.DMA for _ in range(3 * NB)]
    )

    @functools.partial(
        pl.kernel,
        out_type=jax.ShapeDtypeStruct((NC, N_NODES, D), jnp.float32),
        mesh=mesh,
        scratch_types=scratch,
    )
    def k(edges_hbm, x_hbm, src_hbm, dst_hbm, z_hbm, out_hbm, *scr):
        srcv = scr[0:NB]
        dstv = scr[NB:2 * NB]
        ev = scr[2 * NB:3 * NB]
        xv = scr[3 * NB:4 * NB]
        fsh = scr[4 * NB]
        sem_i = scr[4 * NB + 1: 4 * NB + 1 + NB]
        sem_g = scr[4 * NB + 1 + NB: 4 * NB + 1 + 2 * NB]
        sem_d = scr[4 * NB + 1 + 2 * NB: 4 * NB + 1 + 3 * NB]

        cid = lax.axis_index("c")
        sid = lax.axis_index("s")
        wid = sid * NC + cid
        base = wid * EPW

        r0 = sid * RPT
        pltpu.sync_copy(z_hbm.at[pl.ds(r0, RPT)], fsh.at[pl.ds(r0, RPT)])

        @pl.when(sid == NS - 1)
        def _():
            pltpu.sync_copy(z_hbm.at[pl.ds(TAIL0, TAIL)], fsh.at[pl.ds(TAIL0, TAIL)])

        plsc.subcore_barrier()

        def phase1(kk, s):
            off = base + kk * C
            pltpu.async_copy(src_hbm.at[pl.ds(off, C)], srcv[s], sem_i[s])
            pltpu.async_copy(dst_hbm.at[pl.ds(off, C)], dstv[s], sem_i[s])
            pltpu.async_copy(edges_hbm.at[pl.ds(off, C)], ev[s], sem_g[s])

        def drain_i(s):
            pltpu.make_async_copy(src_hbm.at[pl.ds(0, C)], srcv[s], sem_i[s]).wait()
            pltpu.make_async_copy(src_hbm.at[pl.ds(0, C)], dstv[s], sem_i[s]).wait()

        def drain_g(s):
            pltpu.make_async_copy(edges_hbm.at[pl.ds(0, C)], ev[s], sem_g[s]).wait()
            pltpu.make_async_copy(edges_hbm.at[pl.ds(0, C)], xv[s], sem_g[s]).wait()

        def drain_d(s):
            pltpu.make_async_copy(edges_hbm.at[pl.ds(0, C)], ev[s], sem_d[s]).wait()

        def gather(s):
            pltpu.async_copy(x_hbm.at[srcv[s]], xv[s], sem_g[s])

        def position(kk, b, static):
            s_pre = (b + LA) % NB
            s_nxt = (b + 1) % NB
            when = (lambda c: pl.when(bool(c))) if static else pl.when

            @when(kk >= NB - LA)
            def _():
                drain_d(s_pre)

            @when(kk + LA < CHUNKS)
            def _():
                phase1(kk + LA, s_pre)

            @when(kk + 1 < CHUNKS)
            def _():
                drain_i(s_nxt)
                gather(s_nxt)

            drain_g(b)

            @plsc.parallel_loop(0, C, 1, unroll=2)
            def _(i):
                for j in range(D // 16):
                    sl = pl.ds(j * 16, 16)
                    ev[b][i, sl] = ev[b][i, sl] * xv[b][i, sl]

            pltpu.async_copy(ev[b], fsh.at[dstv[b]], sem_d[b], add=True)

        phase1(0, 0)
        phase1(1, 1)
        drain_i(0)
        gather(0)

        def outer(go, carry):
            for b in range(NB):
                position(go + b, b, static=False)
            return carry

        lax.fori_loop(0, BODY // NB, lambda g, c: outer(g * NB, c), 0)

        for kk in range(BODY, CHUNKS):
            position(kk, kk % NB, static=True)
        for kk in range(CHUNKS - LA, CHUNKS):
            drain_d(kk % NB)

        plsc.subcore_barrier()
        pltpu.sync_copy(fsh.at[pl.ds(r0, RPT)], out_hbm.at[cid, pl.ds(r0, RPT)])

        @pl.when(sid == NS - 1)
        def _():
            pltpu.sync_copy(fsh.at[pl.ds(TAIL0, TAIL)],
                            out_hbm.at[cid, pl.ds(TAIL0, TAIL)])

    return k(edges, x, src, dst, zeros)


ER = 2560


def _tc_edge(ea, Wu1, bu1, Wu2, bu2):
    def body(ea_ref, w1_ref, b1_ref, w2_ref, b2_ref, edges_ref):
        a = ea_ref[...]
        h = jnp.tanh(jnp.dot(a.astype(jnp.bfloat16), w1_ref[...].astype(jnp.bfloat16),
                             preferred_element_type=jnp.float32) + b1_ref[...])
        edges_ref[...] = a + jnp.dot(
            h.astype(jnp.bfloat16), w2_ref[...].astype(jnp.bfloat16),
            preferred_element_type=jnp.float32) + b2_ref[...]

    grid = (N_EDGES // ER,)
    return pl.pallas_call(
        body,
        grid=grid,
        in_specs=[
            pl.BlockSpec((ER, D), lambda i: (i, 0)),
            pl.BlockSpec((D, H), lambda i: (0, 0)),
            pl.BlockSpec((1, H), lambda i: (0, 0)),
            pl.BlockSpec((H, D), lambda i: (0, 0)),
            pl.BlockSpec((1, D), lambda i: (0, 0)),
        ],
        out_specs=pl.BlockSpec((ER, D), lambda i: (i, 0)),
        out_shape=jax.ShapeDtypeStruct((N_EDGES, D), jnp.float32),
    )(ea, Wu1, bu1, Wu2, bu2)


NR = 2000


def _tc_node(x, fp, Wg1, bg1, Wg2, bg2):
    def body(x_ref, fp_ref, w1_ref, b1_ref, w2_ref, b2_ref, out_ref):
        f = fp_ref[0] + fp_ref[1]
        h = jnp.tanh(jnp.dot(f.astype(jnp.bfloat16), w1_ref[...].astype(jnp.bfloat16),
                             preferred_element_type=jnp.float32) + b1_ref[...])
        out_ref[...] = x_ref[...] + jnp.dot(
            h.astype(jnp.bfloat16), w2_ref[...].astype(jnp.bfloat16),
            preferred_element_type=jnp.float32) + b2_ref[...]

    grid = (N_NODES // NR,)
    return pl.pallas_call(
        body,
        grid=grid,
        in_specs=[
            pl.BlockSpec((NR, D), lambda i: (i, 0)),
            pl.BlockSpec((NC, NR, D), lambda i: (0, i, 0)),
            pl.BlockSpec((D, H), lambda i: (0, 0)),
            pl.BlockSpec((1, H), lambda i: (0, 0)),
            pl.BlockSpec((H, D), lambda i: (0, 0)),
            pl.BlockSpec((1, D), lambda i: (0, 0)),
        ],
        out_specs=pl.BlockSpec((NR, D), lambda i: (i, 0)),
        out_shape=jax.ShapeDtypeStruct((N_NODES, D), jnp.float32),
    )(x, fp, Wg1, bg1, Wg2, bg2)


def kernel(x, edge_attr, edge_index, Wu1, bu1, Wu2, bu2, Wg1, bg1, Wg2, bg2):
    src = edge_index[0].astype(jnp.int32)
    dst = edge_index[1].astype(jnp.int32)

    edges = _tc_edge(edge_attr, Wu1, bu1.reshape(1, H), Wu2, bu2.reshape(1, D))
    zeros = jnp.zeros((N_NODES, D), jnp.float32)
    fp = _sc_fused(edges, x, src, dst, zeros)
    nodes = _tc_node(x, fp, Wg1, bg1.reshape(1, H), Wg2, bg2.reshape(1, D))
    return nodes, edges


# speedup vs baseline: 5.3951x; 1.1301x over previous
import functools

import jax
import jax.numpy as jnp
from jax import lax
from jax.experimental import pallas as pl
from jax.experimental.pallas import tpu as pltpu
from jax.experimental.pallas import tpu_sc as plsc

N_NODES = 10000
N_EDGES = 320000
D = 128
H = 256

NC = 2
NS = 16
NW = NC * NS
EPW = N_EDGES // NW
C = 40
CHUNKS = EPW // C
RPT = 624
TAIL0 = RPT * NS
TAIL = N_NODES - TAIL0

NB = 4
LA = 2
BODY = (CHUNKS // NB) * NB


def _sc_fused(edges, x, src, dst, zeros):
    mesh = plsc.VectorSubcoreMesh(core_axis_name="c", subcore_axis_name="s")

    scratch = (
        [pltpu.VMEM((C,), jnp.int32) for _ in range(2 * NB)]
        + [pltpu.VMEM((C, D), jnp.float32) for _ in range(2 * NB)]
        + [pltpu.VMEM_SHARED((N_NODES, D), jnp.float32)]
        + [pltpu.SemaphoreType.DMA for _ in range(3 * NB)]
    )

    @functools.partial(
        pl.kernel,
        out_type=jax.ShapeDtypeStruct((NC, N_NODES, D), jnp.float32),
        mesh=mesh,
        scratch_types=scratch,
    )
    def k(edges_hbm, x_hbm, src_hbm, dst_hbm, z_hbm, out_hbm, *scr):
        srcv = scr[0:NB]
        dstv = scr[NB:2 * NB]
        ev = scr[2 * NB:3 * NB]
        xv = scr[3 * NB:4 * NB]
        fsh = scr[4 * NB]
        sem_i = scr[4 * NB + 1: 4 * NB + 1 + NB]
        sem_g = scr[4 * NB + 1 + NB: 4 * NB + 1 + 2 * NB]
        sem_d = scr[4 * NB + 1 + 2 * NB: 4 * NB + 1 + 3 * NB]

        cid = lax.axis_index("c")
        sid = lax.axis_index("s")
        wid = sid * NC + cid
        base = wid * EPW

        r0 = sid * RPT
        pltpu.sync_copy(z_hbm.at[pl.ds(r0, RPT)], fsh.at[pl.ds(r0, RPT)])

        @pl.when(sid == NS - 1)
        def _():
            pltpu.sync_copy(z_hbm.at[pl.ds(TAIL0, TAIL)], fsh.at[pl.ds(TAIL0, TAIL)])

        plsc.subcore_barrier()

        def phase1(kk, s):
            off = base + kk * C
            pltpu.async_copy(src_hbm.at[pl.ds(off, C)], srcv[s], sem_i[s])
            pltpu.async_copy(dst_hbm.at[pl.ds(off, C)], dstv[s], sem_i[s])
            pltpu.async_copy(edges_hbm.at[pl.ds(off, C)], ev[s], sem_g[s])

        def drain_i(s):
            pltpu.make_async_copy(src_hbm.at[pl.ds(0, C)], srcv[s], sem_i[s]).wait()
            pltpu.make_async_copy(src_hbm.at[pl.ds(0, C)], dstv[s], sem_i[s]).wait()

        def drain_g(s):
            pltpu.make_async_copy(edges_hbm.at[pl.ds(0, C)], ev[s], sem_g[s]).wait()
            pltpu.make_async_copy(edges_hbm.at[pl.ds(0, C)], xv[s], sem_g[s]).wait()

        def drain_d(s):
            pltpu.make_async_copy(edges_hbm.at[pl.ds(0, C)], ev[s], sem_d[s]).wait()

        def gather(s):
            pltpu.async_copy(x_hbm.at[srcv[s]], xv[s], sem_g[s])

        def position(kk, b, static):
            s_pre = (b + LA) % NB
            s_nxt = (b + 1) % NB
            when = (lambda c: pl.when(bool(c))) if static else pl.when

            @when(kk >= NB - LA)
            def _():
                drain_d(s_pre)

            @when(kk + LA < CHUNKS)
            def _():
                phase1(kk + LA, s_pre)

            @when(kk + 1 < CHUNKS)
            def _():
                drain_i(s_nxt)
                gather(s_nxt)

            drain_g(b)

            @plsc.parallel_loop(0, C, 1, unroll=2)
            def _(i):
                for j in range(D // 16):
                    sl = pl.ds(j * 16, 16)
                    ev[b][i, sl] = ev[b][i, sl] * xv[b][i, sl]

            pltpu.async_copy(ev[b], fsh.at[dstv[b]], sem_d[b], add=True)

        phase1(0, 0)
        phase1(1, 1)
        drain_i(0)
        gather(0)

        def outer(go, carry):
            for b in range(NB):
                position(go + b, b, static=False)
            return carry

        lax.fori_loop(0, BODY // NB, lambda g, c: outer(g * NB, c), 0)

        for kk in range(BODY, CHUNKS):
            position(kk, kk % NB, static=True)
        for kk in range(CHUNKS - LA, CHUNKS):
            drain_d(kk % NB)

        plsc.subcore_barrier()
        pltpu.sync_copy(fsh.at[pl.ds(r0, RPT)], out_hbm.at[cid, pl.ds(r0, RPT)])

        @pl.when(sid == NS - 1)
        def _():
            pltpu.sync_copy(fsh.at[pl.ds(TAIL0, TAIL)],
                            out_hbm.at[cid, pl.ds(TAIL0, TAIL)])

    return k(edges, x, src, dst, zeros)


ER = 6400


def _tc_edge(ea, Wu1, bu1, Wu2, bu2):
    def body(ea_ref, w1_ref, b1_ref, w2_ref, b2_ref, edges_ref):
        a = ea_ref[...]
        h = jnp.tanh(jnp.dot(a.astype(jnp.bfloat16), w1_ref[...].astype(jnp.bfloat16),
                             preferred_element_type=jnp.float32) + b1_ref[...])
        edges_ref[...] = a + jnp.dot(
            h.astype(jnp.bfloat16), w2_ref[...].astype(jnp.bfloat16),
            preferred_element_type=jnp.float32) + b2_ref[...]

    grid = (N_EDGES // ER,)
    return pl.pallas_call(
        body,
        grid=grid,
        in_specs=[
            pl.BlockSpec((ER, D), lambda i: (i, 0)),
            pl.BlockSpec((D, H), lambda i: (0, 0)),
            pl.BlockSpec((1, H), lambda i: (0, 0)),
            pl.BlockSpec((H, D), lambda i: (0, 0)),
            pl.BlockSpec((1, D), lambda i: (0, 0)),
        ],
        out_specs=pl.BlockSpec((ER, D), lambda i: (i, 0)),
        out_shape=jax.ShapeDtypeStruct((N_EDGES, D), jnp.float32),
    )(ea, Wu1, bu1, Wu2, bu2)


NR = 2000


def _tc_node(x, fp, Wg1, bg1, Wg2, bg2):
    def body(x_ref, fp_ref, w1_ref, b1_ref, w2_ref, b2_ref, out_ref):
        f = fp_ref[0] + fp_ref[1]
        h = jnp.tanh(jnp.dot(f.astype(jnp.bfloat16), w1_ref[...].astype(jnp.bfloat16),
                             preferred_element_type=jnp.float32) + b1_ref[...])
        out_ref[...] = x_ref[...] + jnp.dot(
            h.astype(jnp.bfloat16), w2_ref[...].astype(jnp.bfloat16),
            preferred_element_type=jnp.float32) + b2_ref[...]

    grid = (N_NODES // NR,)
    return pl.pallas_call(
        body,
        grid=grid,
        in_specs=[
            pl.BlockSpec((NR, D), lambda i: (i, 0)),
            pl.BlockSpec((NC, NR, D), lambda i: (0, i, 0)),
            pl.BlockSpec((D, H), lambda i: (0, 0)),
            pl.BlockSpec((1, H), lambda i: (0, 0)),
            pl.BlockSpec((H, D), lambda i: (0, 0)),
            pl.BlockSpec((1, D), lambda i: (0, 0)),
        ],
        out_specs=pl.BlockSpec((NR, D), lambda i: (i, 0)),
        out_shape=jax.ShapeDtypeStruct((N_NODES, D), jnp.float32),
    )(x, fp, Wg1, bg1, Wg2, bg2)


def kernel(x, edge_attr, edge_index, Wu1, bu1, Wu2, bu2, Wg1, bg1, Wg2, bg2):
    src = edge_index[0].astype(jnp.int32)
    dst = edge_index[1].astype(jnp.int32)

    edges = _tc_edge(edge_attr, Wu1, bu1.reshape(1, H), Wu2, bu2.reshape(1, D))
    zeros = jnp.zeros((N_NODES, D), jnp.float32)
    fp = _sc_fused(edges, x, src, dst, zeros)
    nodes = _tc_node(x, fp, Wg1, bg1.reshape(1, H), Wg2, bg2.reshape(1, D))
    return nodes, edges


# speedup vs baseline: 5.5708x; 1.0326x over previous
import functools

import jax
import jax.numpy as jnp
from jax import lax
from jax.experimental import pallas as pl
from jax.experimental.pallas import tpu as pltpu
from jax.experimental.pallas import tpu_sc as plsc

N_NODES = 10000
N_EDGES = 320000
D = 128
H = 256

NC = 2
NS = 16
NW = NC * NS
EPW = N_EDGES // NW
C = 40
CHUNKS = EPW // C
RPT = 624
TAIL0 = RPT * NS
TAIL = N_NODES - TAIL0

NB = 4
LA = 2
BODY = (CHUNKS // NB) * NB


def _sc_fused(edges, x, src, dst):
    mesh = plsc.VectorSubcoreMesh(core_axis_name="c", subcore_axis_name="s")

    scratch = (
        [pltpu.VMEM((C,), jnp.int32) for _ in range(2 * NB)]
        + [pltpu.VMEM((C, D), jnp.float32) for _ in range(2 * NB)]
        + [pltpu.VMEM_SHARED((N_NODES, D), jnp.float32)]
        + [pltpu.SemaphoreType.DMA for _ in range(3 * NB)]
    )

    @functools.partial(
        pl.kernel,
        out_type=jax.ShapeDtypeStruct((NC, N_NODES, D), jnp.float32),
        mesh=mesh,
        scratch_types=scratch,
    )
    def k(edges_hbm, x_hbm, src_hbm, dst_hbm, out_hbm, *scr):
        srcv = scr[0:NB]
        dstv = scr[NB:2 * NB]
        ev = scr[2 * NB:3 * NB]
        xv = scr[3 * NB:4 * NB]
        fsh = scr[4 * NB]
        sem_i = scr[4 * NB + 1: 4 * NB + 1 + NB]
        sem_g = scr[4 * NB + 1 + NB: 4 * NB + 1 + 2 * NB]
        sem_d = scr[4 * NB + 1 + 2 * NB: 4 * NB + 1 + 3 * NB]

        cid = lax.axis_index("c")
        sid = lax.axis_index("s")
        wid = sid * NC + cid
        base = wid * EPW

        @plsc.parallel_loop(0, C, 1, unroll=2)
        def _(i):
            for j in range(D // 16):
                ev[0][i, pl.ds(j * 16, 16)] = jnp.zeros((16,), jnp.float32)

        r0 = sid * RPT
        for t in range(RPT // C):
            pltpu.sync_copy(ev[0], fsh.at[pl.ds(r0 + t * C, C)])
        pltpu.sync_copy(ev[0].at[pl.ds(0, RPT - (RPT // C) * C)],
                        fsh.at[pl.ds(r0 + (RPT // C) * C, RPT - (RPT // C) * C)])

        @pl.when(sid == NS - 1)
        def _():
            pltpu.sync_copy(ev[0].at[pl.ds(0, TAIL)], fsh.at[pl.ds(TAIL0, TAIL)])

        plsc.subcore_barrier()

        def phase1(kk, s):
            off = base + kk * C
            pltpu.async_copy(src_hbm.at[pl.ds(off, C)], srcv[s], sem_i[s])
            pltpu.async_copy(dst_hbm.at[pl.ds(off, C)], dstv[s], sem_i[s])
            pltpu.async_copy(edges_hbm.at[pl.ds(off, C)], ev[s], sem_g[s])

        def drain_i(s):
            pltpu.make_async_copy(src_hbm.at[pl.ds(0, C)], srcv[s], sem_i[s]).wait()
            pltpu.make_async_copy(src_hbm.at[pl.ds(0, C)], dstv[s], sem_i[s]).wait()

        def drain_g(s):
            pltpu.make_async_copy(edges_hbm.at[pl.ds(0, C)], ev[s], sem_g[s]).wait()
            pltpu.make_async_copy(edges_hbm.at[pl.ds(0, C)], xv[s], sem_g[s]).wait()

        def drain_d(s):
            pltpu.make_async_copy(edges_hbm.at[pl.ds(0, C)], ev[s], sem_d[s]).wait()

        def gather(s):
            pltpu.async_copy(x_hbm.at[srcv[s]], xv[s], sem_g[s])

        def position(kk, b, static):
            s_pre = (b + LA) % NB
            s_nxt = (b + 1) % NB
            when = (lambda c: pl.when(bool(c))) if static else pl.when

            @when(kk >= NB - LA)
            def _():
                drain_d(s_pre)

            @when(kk + LA < CHUNKS)
            def _():
                phase1(kk + LA, s_pre)

            @when(kk + 1 < CHUNKS)
            def _():
                drain_i(s_nxt)
                gather(s_nxt)

            drain_g(b)

            @plsc.parallel_loop(0, C, 1, unroll=2)
            def _(i):
                for j in range(D // 16):
                    sl = pl.ds(j * 16, 16)
                    ev[b][i, sl] = ev[b][i, sl] * xv[b][i, sl]

            pltpu.async_copy(ev[b], fsh.at[dstv[b]], sem_d[b], add=True)

        phase1(0, 0)
        phase1(1, 1)
        drain_i(0)
        gather(0)

        def outer(go, carry):
            for b in range(NB):
                position(go + b, b, static=False)
            return carry

        lax.fori_loop(0, BODY // NB, lambda g, c: outer(g * NB, c), 0)

        for kk in range(BODY, CHUNKS):
            position(kk, kk % NB, static=True)
        for kk in range(CHUNKS - LA, CHUNKS):
            drain_d(kk % NB)

        plsc.subcore_barrier()
        pltpu.sync_copy(fsh.at[pl.ds(r0, RPT)], out_hbm.at[cid, pl.ds(r0, RPT)])

        @pl.when(sid == NS - 1)
        def _():
            pltpu.sync_copy(fsh.at[pl.ds(TAIL0, TAIL)],
                            out_hbm.at[cid, pl.ds(TAIL0, TAIL)])

    return k(edges, x, src, dst)


ER = 8000


def _tc_edge(ea, Wu1, bu1, Wu2, bu2):
    def body(ea_ref, w1_ref, b1_ref, w2_ref, b2_ref, edges_ref):
        a = ea_ref[...]
        h = jnp.tanh(jnp.dot(a.astype(jnp.bfloat16), w1_ref[...].astype(jnp.bfloat16),
                             preferred_element_type=jnp.float32) + b1_ref[...])
        edges_ref[...] = a + jnp.dot(
            h.astype(jnp.bfloat16), w2_ref[...].astype(jnp.bfloat16),
            preferred_element_type=jnp.float32) + b2_ref[...]

    grid = (N_EDGES // ER,)
    return pl.pallas_call(
        body,
        grid=grid,
        in_specs=[
            pl.BlockSpec((ER, D), lambda i: (i, 0)),
            pl.BlockSpec((D, H), lambda i: (0, 0)),
            pl.BlockSpec((1, H), lambda i: (0, 0)),
            pl.BlockSpec((H, D), lambda i: (0, 0)),
            pl.BlockSpec((1, D), lambda i: (0, 0)),
        ],
        out_specs=pl.BlockSpec((ER, D), lambda i: (i, 0)),
        out_shape=jax.ShapeDtypeStruct((N_EDGES, D), jnp.float32),
    )(ea, Wu1, bu1, Wu2, bu2)


NR = 2000


def _tc_node(x, fp, Wg1, bg1, Wg2, bg2):
    def body(x_ref, fp_ref, w1_ref, b1_ref, w2_ref, b2_ref, out_ref):
        f = fp_ref[0] + fp_ref[1]
        h = jnp.tanh(jnp.dot(f.astype(jnp.bfloat16), w1_ref[...].astype(jnp.bfloat16),
                             preferred_element_type=jnp.float32) + b1_ref[...])
        out_ref[...] = x_ref[...] + jnp.dot(
            h.astype(jnp.bfloat16), w2_ref[...].astype(jnp.bfloat16),
            preferred_element_type=jnp.float32) + b2_ref[...]

    grid = (N_NODES // NR,)
    return pl.pallas_call(
        body,
        grid=grid,
        in_specs=[
            pl.BlockSpec((NR, D), lambda i: (i, 0)),
            pl.BlockSpec((NC, NR, D), lambda i: (0, i, 0)),
            pl.BlockSpec((D, H), lambda i: (0, 0)),
            pl.BlockSpec((1, H), lambda i: (0, 0)),
            pl.BlockSpec((H, D), lambda i: (0, 0)),
            pl.BlockSpec((1, D), lambda i: (0, 0)),
        ],
        out_specs=pl.BlockSpec((NR, D), lambda i: (i, 0)),
        out_shape=jax.ShapeDtypeStruct((N_NODES, D), jnp.float32),
    )(x, fp, Wg1, bg1, Wg2, bg2)


def kernel(x, edge_attr, edge_index, Wu1, bu1, Wu2, bu2, Wg1, bg1, Wg2, bg2):
    src = edge_index[0].astype(jnp.int32)
    dst = edge_index[1].astype(jnp.int32)

    edges = _tc_edge(edge_attr, Wu1, bu1.reshape(1, H), Wu2, bu2.reshape(1, D))
    fp = _sc_fused(edges, x, src, dst)
    nodes = _tc_node(x, fp, Wg1, bg1.reshape(1, H), Wg2, bg2.reshape(1, D))
    return nodes, edges


# speedup vs baseline: 5.7339x; 1.0293x over previous
import functools

import jax
import jax.numpy as jnp
from jax import lax
from jax.experimental import pallas as pl
from jax.experimental.pallas import tpu as pltpu
from jax.experimental.pallas import tpu_sc as plsc

N_NODES = 10000
N_EDGES = 320000
D = 128
H = 256

NC = 2
NS = 16
NW = NC * NS
EPW = N_EDGES // NW
C = 40
CHUNKS = EPW // C
RPT = 624
TAIL0 = RPT * NS
TAIL = N_NODES - TAIL0

NB = 4
LA = 2
BODY = (CHUNKS // NB) * NB


def _sc_fused(edges, x, src, dst):
    mesh = plsc.VectorSubcoreMesh(core_axis_name="c", subcore_axis_name="s")

    scratch = (
        [pltpu.VMEM((C,), jnp.int32) for _ in range(2 * NB)]
        + [pltpu.VMEM((C, D), jnp.float32) for _ in range(2 * NB)]
        + [pltpu.VMEM_SHARED((N_NODES, D), jnp.float32)]
        + [pltpu.SemaphoreType.DMA for _ in range(3 * NB)]
    )

    @functools.partial(
        pl.kernel,
        out_type=jax.ShapeDtypeStruct((NC, N_NODES, D), jnp.float32),
        mesh=mesh,
        scratch_types=scratch,
    )
    def k(edges_hbm, x_hbm, src_hbm, dst_hbm, out_hbm, *scr):
        srcv = scr[0:NB]
        dstv = scr[NB:2 * NB]
        ev = scr[2 * NB:3 * NB]
        xv = scr[3 * NB:4 * NB]
        fsh = scr[4 * NB]
        sem_i = scr[4 * NB + 1: 4 * NB + 1 + NB]
        sem_g = scr[4 * NB + 1 + NB: 4 * NB + 1 + 2 * NB]
        sem_d = scr[4 * NB + 1 + 2 * NB: 4 * NB + 1 + 3 * NB]

        cid = lax.axis_index("c")
        sid = lax.axis_index("s")
        wid = sid * NC + cid
        base = wid * EPW

        @plsc.parallel_loop(0, C, 1, unroll=2)
        def _(i):
            for j in range(D // 16):
                ev[0][i, pl.ds(j * 16, 16)] = jnp.zeros((16,), jnp.float32)

        r0 = sid * RPT
        for t in range(RPT // C):
            pltpu.sync_copy(ev[0], fsh.at[pl.ds(r0 + t * C, C)])
        pltpu.sync_copy(ev[0].at[pl.ds(0, RPT - (RPT // C) * C)],
                        fsh.at[pl.ds(r0 + (RPT // C) * C, RPT - (RPT // C) * C)])

        @pl.when(sid == NS - 1)
        def _():
            pltpu.sync_copy(ev[0].at[pl.ds(0, TAIL)], fsh.at[pl.ds(TAIL0, TAIL)])

        plsc.subcore_barrier()

        def phase1(kk, s):
            off = base + kk * C
            pltpu.async_copy(src_hbm.at[pl.ds(off, C)], srcv[s], sem_i[s])
            pltpu.async_copy(dst_hbm.at[pl.ds(off, C)], dstv[s], sem_i[s])
            pltpu.async_copy(edges_hbm.at[pl.ds(off, C)], ev[s], sem_g[s])

        def drain_i(s):
            pltpu.make_async_copy(src_hbm.at[pl.ds(0, C)], srcv[s], sem_i[s]).wait()
            pltpu.make_async_copy(src_hbm.at[pl.ds(0, C)], dstv[s], sem_i[s]).wait()

        def drain_g(s):
            pltpu.make_async_copy(edges_hbm.at[pl.ds(0, C)], ev[s], sem_g[s]).wait()
            pltpu.make_async_copy(edges_hbm.at[pl.ds(0, C)], xv[s], sem_g[s]).wait()

        def drain_d(s):
            pltpu.make_async_copy(edges_hbm.at[pl.ds(0, C)], ev[s], sem_d[s]).wait()

        def gather(s):
            pltpu.async_copy(x_hbm.at[srcv[s]], xv[s], sem_g[s])

        def position(kk, b, static):
            s_pre = (b + LA) % NB
            s_nxt = (b + 1) % NB
            when = (lambda c: pl.when(bool(c))) if static else pl.when

            @when(kk >= NB - LA)
            def _():
                drain_d(s_pre)

            @when(kk + LA < CHUNKS)
            def _():
                phase1(kk + LA, s_pre)

            @when(kk + 1 < CHUNKS)
            def _():
                drain_i(s_nxt)
                gather(s_nxt)

            drain_g(b)

            @plsc.parallel_loop(0, C, 1, unroll=2)
            def _(i):
                for j in range(D // 16):
                    sl = pl.ds(j * 16, 16)
                    ev[b][i, sl] = ev[b][i, sl] * xv[b][i, sl]

            pltpu.async_copy(ev[b], fsh.at[dstv[b]], sem_d[b], add=True)

        phase1(0, 0)
        phase1(1, 1)
        drain_i(0)
        gather(0)

        def outer(go, carry):
            for b in range(NB):
                position(go + b, b, static=False)
            return carry

        lax.fori_loop(0, BODY // NB, lambda g, c: outer(g * NB, c), 0)

        for kk in range(BODY, CHUNKS):
            position(kk, kk % NB, static=True)
        for kk in range(CHUNKS - LA, CHUNKS):
            drain_d(kk % NB)

        plsc.subcore_barrier()
        pltpu.sync_copy(fsh.at[pl.ds(r0, RPT)], out_hbm.at[cid, pl.ds(r0, RPT)])

        @pl.when(sid == NS - 1)
        def _():
            pltpu.sync_copy(fsh.at[pl.ds(TAIL0, TAIL)],
                            out_hbm.at[cid, pl.ds(TAIL0, TAIL)])

    return k(edges, x, src, dst)


ER = 12800


def _tc_edge(ea, Wu1, bu1, Wu2, bu2):
    def body(ea_ref, w1_ref, b1_ref, w2_ref, b2_ref, edges_ref):
        a = ea_ref[...]
        h = jnp.tanh(jnp.dot(a.astype(jnp.bfloat16), w1_ref[...].astype(jnp.bfloat16),
                             preferred_element_type=jnp.float32) + b1_ref[...])
        edges_ref[...] = a + jnp.dot(
            h.astype(jnp.bfloat16), w2_ref[...].astype(jnp.bfloat16),
            preferred_element_type=jnp.float32) + b2_ref[...]

    grid = (N_EDGES // ER,)
    return pl.pallas_call(
        body,
        grid=grid,
        in_specs=[
            pl.BlockSpec((ER, D), lambda i: (i, 0)),
            pl.BlockSpec((D, H), lambda i: (0, 0)),
            pl.BlockSpec((1, H), lambda i: (0, 0)),
            pl.BlockSpec((H, D), lambda i: (0, 0)),
            pl.BlockSpec((1, D), lambda i: (0, 0)),
        ],
        out_specs=pl.BlockSpec((ER, D), lambda i: (i, 0)),
        out_shape=jax.ShapeDtypeStruct((N_EDGES, D), jnp.float32),
    )(ea, Wu1, bu1, Wu2, bu2)


NR = 2000


def _tc_node(x, fp, Wg1, bg1, Wg2, bg2):
    def body(x_ref, fp_ref, w1_ref, b1_ref, w2_ref, b2_ref, out_ref):
        f = fp_ref[0] + fp_ref[1]
        h = jnp.tanh(jnp.dot(f.astype(jnp.bfloat16), w1_ref[...].astype(jnp.bfloat16),
                             preferred_element_type=jnp.float32) + b1_ref[...])
        out_ref[...] = x_ref[...] + jnp.dot(
            h.astype(jnp.bfloat16), w2_ref[...].astype(jnp.bfloat16),
            preferred_element_type=jnp.float32) + b2_ref[...]

    grid = (N_NODES // NR,)
    return pl.pallas_call(
        body,
        grid=grid,
        in_specs=[
            pl.BlockSpec((NR, D), lambda i: (i, 0)),
            pl.BlockSpec((NC, NR, D), lambda i: (0, i, 0)),
            pl.BlockSpec((D, H), lambda i: (0, 0)),
            pl.BlockSpec((1, H), lambda i: (0, 0)),
            pl.BlockSpec((H, D), lambda i: (0, 0)),
            pl.BlockSpec((1, D), lambda i: (0, 0)),
        ],
        out_specs=pl.BlockSpec((NR, D), lambda i: (i, 0)),
        out_shape=jax.ShapeDtypeStruct((N_NODES, D), jnp.float32),
    )(x, fp, Wg1, bg1, Wg2, bg2)


def kernel(x, edge_attr, edge_index, Wu1, bu1, Wu2, bu2, Wg1, bg1, Wg2, bg2):
    src = edge_index[0].astype(jnp.int32)
    dst = edge_index[1].astype(jnp.int32)

    edges = _tc_edge(edge_attr, Wu1, bu1.reshape(1, H), Wu2, bu2.reshape(1, D))
    fp = _sc_fused(edges, x, src, dst)
    nodes = _tc_node(x, fp, Wg1, bg1.reshape(1, H), Wg2, bg2.reshape(1, D))
    return nodes, edges


# speedup vs baseline: 5.8002x; 1.0116x over previous
import functools

import jax
import jax.numpy as jnp
from jax import lax
from jax.experimental import pallas as pl
from jax.experimental.pallas import tpu as pltpu
from jax.experimental.pallas import tpu_sc as plsc

N_NODES = 10000
N_EDGES = 320000
D = 128
H = 256

NC = 2
NS = 16
NW = NC * NS
EPW = N_EDGES // NW
C = 40
CHUNKS = EPW // C
RPT = 624
TAIL0 = RPT * NS
TAIL = N_NODES - TAIL0

NB = 4
LA = 2
BODY = (CHUNKS // NB) * NB


def _sc_fused(edges, x, src, dst):
    mesh = plsc.VectorSubcoreMesh(core_axis_name="c", subcore_axis_name="s")

    scratch = (
        [pltpu.VMEM((C,), jnp.int32) for _ in range(2 * NB)]
        + [pltpu.VMEM((C, D), jnp.float32) for _ in range(2 * NB)]
        + [pltpu.VMEM_SHARED((N_NODES, D), jnp.float32)]
        + [pltpu.SemaphoreType.DMA for _ in range(3 * NB)]
    )

    @functools.partial(
        pl.kernel,
        out_type=jax.ShapeDtypeStruct((NC, N_NODES, D), jnp.float32),
        mesh=mesh,
        scratch_types=scratch,
    )
    def k(edges_hbm, x_hbm, src_hbm, dst_hbm, out_hbm, *scr):
        srcv = scr[0:NB]
        dstv = scr[NB:2 * NB]
        ev = scr[2 * NB:3 * NB]
        xv = scr[3 * NB:4 * NB]
        fsh = scr[4 * NB]
        sem_i = scr[4 * NB + 1: 4 * NB + 1 + NB]
        sem_g = scr[4 * NB + 1 + NB: 4 * NB + 1 + 2 * NB]
        sem_d = scr[4 * NB + 1 + 2 * NB: 4 * NB + 1 + 3 * NB]

        cid = lax.axis_index("c")
        sid = lax.axis_index("s")
        wid = sid * NC + cid
        base = wid * EPW

        @plsc.parallel_loop(0, C, 1, unroll=2)
        def _(i):
            for j in range(D // 16):
                ev[0][i, pl.ds(j * 16, 16)] = jnp.zeros((16,), jnp.float32)

        r0 = sid * RPT
        for t in range(RPT // C):
            pltpu.sync_copy(ev[0], fsh.at[pl.ds(r0 + t * C, C)])
        pltpu.sync_copy(ev[0].at[pl.ds(0, RPT - (RPT // C) * C)],
                        fsh.at[pl.ds(r0 + (RPT // C) * C, RPT - (RPT // C) * C)])

        @pl.when(sid == NS - 1)
        def _():
            pltpu.sync_copy(ev[0].at[pl.ds(0, TAIL)], fsh.at[pl.ds(TAIL0, TAIL)])

        plsc.subcore_barrier()

        def phase1(kk, s):
            off = base + kk * C
            pltpu.async_copy(src_hbm.at[pl.ds(off, C)], srcv[s], sem_i[s])
            pltpu.async_copy(dst_hbm.at[pl.ds(off, C)], dstv[s], sem_i[s])
            pltpu.async_copy(edges_hbm.at[pl.ds(off, C)], ev[s], sem_g[s])

        def drain_i(s):
            pltpu.make_async_copy(src_hbm.at[pl.ds(0, C)], srcv[s], sem_i[s]).wait()
            pltpu.make_async_copy(src_hbm.at[pl.ds(0, C)], dstv[s], sem_i[s]).wait()

        def drain_g(s):
            pltpu.make_async_copy(edges_hbm.at[pl.ds(0, C)], ev[s], sem_g[s]).wait()
            pltpu.make_async_copy(edges_hbm.at[pl.ds(0, C)], xv[s], sem_g[s]).wait()

        def drain_d(s):
            pltpu.make_async_copy(edges_hbm.at[pl.ds(0, C)], ev[s], sem_d[s]).wait()

        def gather(s):
            pltpu.async_copy(x_hbm.at[srcv[s]], xv[s], sem_g[s])

        def position(kk, b, static):
            s_pre = (b + LA) % NB
            s_nxt = (b + 1) % NB
            when = (lambda c: pl.when(bool(c))) if static else pl.when

            @when(kk >= NB - LA)
            def _():
                drain_d(s_pre)

            @when(kk + LA < CHUNKS)
            def _():
                phase1(kk + LA, s_pre)

            @when(kk + 1 < CHUNKS)
            def _():
                drain_i(s_nxt)
                gather(s_nxt)

            drain_g(b)

            @plsc.parallel_loop(0, C, 1, unroll=4)
            def _(i):
                for j in range(D // 16):
                    sl = pl.ds(j * 16, 16)
                    ev[b][i, sl] = ev[b][i, sl] * xv[b][i, sl]

            pltpu.async_copy(ev[b], fsh.at[dstv[b]], sem_d[b], add=True)

        phase1(0, 0)
        phase1(1, 1)
        drain_i(0)
        gather(0)

        def outer(go, carry):
            for b in range(NB):
                position(go + b, b, static=False)
            return carry

        lax.fori_loop(0, BODY // NB, lambda g, c: outer(g * NB, c), 0)

        for kk in range(BODY, CHUNKS):
            position(kk, kk % NB, static=True)
        for kk in range(CHUNKS - LA, CHUNKS):
            drain_d(kk % NB)

        plsc.subcore_barrier()
        pltpu.sync_copy(fsh.at[pl.ds(r0, RPT)], out_hbm.at[cid, pl.ds(r0, RPT)])

        @pl.when(sid == NS - 1)
        def _():
            pltpu.sync_copy(fsh.at[pl.ds(TAIL0, TAIL)],
                            out_hbm.at[cid, pl.ds(TAIL0, TAIL)])

    return k(edges, x, src, dst)


ER = 16000


def _tc_edge(ea, Wu1, bu1, Wu2, bu2):
    def body(ea_ref, w1_ref, b1_ref, w2_ref, b2_ref, edges_ref):
        a = ea_ref[...]
        h = jnp.tanh(jnp.dot(a.astype(jnp.bfloat16), w1_ref[...].astype(jnp.bfloat16),
                             preferred_element_type=jnp.float32) + b1_ref[...])
        edges_ref[...] = a + jnp.dot(
            h.astype(jnp.bfloat16), w2_ref[...].astype(jnp.bfloat16),
            preferred_element_type=jnp.float32) + b2_ref[...]

    grid = (N_EDGES // ER,)
    return pl.pallas_call(
        body,
        grid=grid,
        in_specs=[
            pl.BlockSpec((ER, D), lambda i: (i, 0)),
            pl.BlockSpec((D, H), lambda i: (0, 0)),
            pl.BlockSpec((1, H), lambda i: (0, 0)),
            pl.BlockSpec((H, D), lambda i: (0, 0)),
            pl.BlockSpec((1, D), lambda i: (0, 0)),
        ],
        out_specs=pl.BlockSpec((ER, D), lambda i: (i, 0)),
        out_shape=jax.ShapeDtypeStruct((N_EDGES, D), jnp.float32),
    )(ea, Wu1, bu1, Wu2, bu2)


NR = 2000


def _tc_node(x, fp, Wg1, bg1, Wg2, bg2):
    def body(x_ref, fp_ref, w1_ref, b1_ref, w2_ref, b2_ref, out_ref):
        f = fp_ref[0] + fp_ref[1]
        h = jnp.tanh(jnp.dot(f.astype(jnp.bfloat16), w1_ref[...].astype(jnp.bfloat16),
                             preferred_element_type=jnp.float32) + b1_ref[...])
        out_ref[...] = x_ref[...] + jnp.dot(
            h.astype(jnp.bfloat16), w2_ref[...].astype(jnp.bfloat16),
            preferred_element_type=jnp.float32) + b2_ref[...]

    grid = (N_NODES // NR,)
    return pl.pallas_call(
        body,
        grid=grid,
        in_specs=[
            pl.BlockSpec((NR, D), lambda i: (i, 0)),
            pl.BlockSpec((NC, NR, D), lambda i: (0, i, 0)),
            pl.BlockSpec((D, H), lambda i: (0, 0)),
            pl.BlockSpec((1, H), lambda i: (0, 0)),
            pl.BlockSpec((H, D), lambda i: (0, 0)),
            pl.BlockSpec((1, D), lambda i: (0, 0)),
        ],
        out_specs=pl.BlockSpec((NR, D), lambda i: (i, 0)),
        out_shape=jax.ShapeDtypeStruct((N_NODES, D), jnp.float32),
    )(x, fp, Wg1, bg1, Wg2, bg2)


def kernel(x, edge_attr, edge_index, Wu1, bu1, Wu2, bu2, Wg1, bg1, Wg2, bg2):
    src = edge_index[0].astype(jnp.int32)
    dst = edge_index[1].astype(jnp.int32)

    edges = _tc_edge(edge_attr, Wu1, bu1.reshape(1, H), Wu2, bu2.reshape(1, D))
    fp = _sc_fused(edges, x, src, dst)
    nodes = _tc_node(x, fp, Wg1, bg1.reshape(1, H), Wg2, bg2.reshape(1, D))
    return nodes, edges
